# Initial kernel scaffold; baseline (speedup 1.0000x reference)
import jax
import jax.numpy as jnp
from jax.experimental import pallas as pl


def kernel(edge_tokens, question_tokens, edge_batch, selected_mask, selection_order, order_emb, type_emb, gn_g, gn_b, qf_g, qf_b, qf_W, qf_bias, m1_g, m1_b, m1_W, m1_bias, m2_g, m2_b, m2_W, m2_bias, lh_g, lh_b, lh_W1, lh_b1, lh_W2, lh_b2, sp_g, sp_b, sp_W1, sp_b1, sp_W2, sp_b2):
    raise NotImplementedError("write your pallas kernel here")


# speedup vs baseline: 5.3451x; 5.3451x over previous
import jax
import jax.numpy as jnp
from jax import lax
from jax.experimental import pallas as pl

_SQRT2 = 1.4142135623730951


def _gelu(x):
    return x * 0.5 * (1.0 + lax.erf(x / _SQRT2))


def _ln(x, g, b):
    m = jnp.mean(x, axis=-1, keepdims=True)
    d = x - m
    v = jnp.mean(d * d, axis=-1, keepdims=True)
    return d * lax.rsqrt(v + 1e-5) * g + b


def _dot(a, b):
    return jnp.dot(a, b, preferred_element_type=jnp.float32)


def _pick_block(e):
    for b in (2000, 1600, 1280, 1024, 1000, 800, 640, 512, 256, 128, 64, 32, 16, 8):
        if e % b == 0:
            return b
    return e


def _qf_kernel(q_ref, qfg_ref, qfb_ref, qfW_ref, qfbias_ref, type2_ref, out_ref):
    q = q_ref[...]
    y = _dot(_ln(q, qfg_ref[...], qfb_ref[...]), qfW_ref[...]) + qfbias_ref[...]
    out_ref[...] = _gelu(y) + type2_ref[...]


def _edge_fwd_kernel(et_ref, eb_ref, ord_ref, mask_ref, qf2_ref, ordemb_ref,
                     m1g_ref, m1b_ref, m1W_ref, m1bias_ref,
                     m2g_ref, m2b_ref, m2W_ref, m2bias_ref,
                     x_ref, ssum_ref, cnt_ref):
    i = pl.program_id(0)
    B = et_ref.shape[0]
    G = qf2_ref.shape[0]
    R = ordemb_ref.shape[0]

    eb = eb_ref[0, 0, :]
    P = (eb[:, None] == lax.broadcasted_iota(jnp.int32, (B, G), 1)).astype(jnp.float32)
    oidx = jnp.clip(ord_ref[0, 0, :], -1, R - 2) + 1
    Po = (oidx[:, None] == lax.broadcasted_iota(jnp.int32, (B, R), 1)).astype(jnp.float32)

    et = et_ref[...] + _dot(Po, ordemb_ref[...]) + _dot(P, qf2_ref[...])
    x = _gelu(_dot(_ln(et, m1g_ref[...], m1b_ref[...]), m1W_ref[...]) + m1bias_ref[...])
    x = _gelu(_dot(_ln(x, m2g_ref[...], m2b_ref[...]), m2W_ref[...]) + m2bias_ref[...])
    x_ref[...] = x

    mf = mask_ref[0, 0, :].astype(jnp.float32)
    xm = x * mf[:, None]
    psum = lax.dot_general(P, xm, (((0,), (0,)), ((), ())),
                           preferred_element_type=jnp.float32)
    pcnt = lax.dot_general(P, mf[:, None], (((0,), (0,)), ((), ())),
                           preferred_element_type=jnp.float32)

    @pl.when(i == 0)
    def _init():
        ssum_ref[...] = jnp.zeros_like(ssum_ref)
        cnt_ref[...] = jnp.zeros_like(cnt_ref)

    ssum_ref[...] += psum
    cnt_ref[...] += pcnt


def _graph_kernel(ssum_ref, cnt_ref, q_ref, gng_ref, gnb_ref,
                  spg_ref, spb_ref, spW1_ref, spb1_ref, spW2_ref, spb2_ref,
                  cs_ref, stop_ref, tab_ref, r_ref):
    ssum = ssum_ref[...]
    q = q_ref[...]
    cnt_raw = cnt_ref[...]
    cnt_c = jnp.maximum(cnt_raw, 1.0)
    cs = _ln(ssum / cnt_c + q, gng_ref[...], gnb_ref[...])
    cs_ref[...] = cs
    r = 1.0 / (cnt_raw + 1.0)
    r_ref[...] = r
    tab_ref[...] = jnp.concatenate([cs, ssum * r + q], axis=1)
    catg = jnp.concatenate([cs, q], axis=1)
    hs = _gelu(_dot(_ln(catg, spg_ref[...], spb_ref[...]), spW1_ref[...]) + spb1_ref[...])
    stop_ref[...] = _dot(hs, spW2_ref[...]) + spb2_ref[...]


def _edge_head_kernel(x_ref, eb_ref, tab_ref, r_ref, gng_ref, gnb_ref,
                      lhg_ref, lhb_ref, lhW1_ref, lhb1_ref, lhW2_ref, lhb2_ref,
                      out_ref):
    B = x_ref.shape[0]
    G = tab_ref.shape[0]
    H = x_ref.shape[1]

    eb = eb_ref[0, 0, :]
    P = (eb[:, None] == lax.broadcasted_iota(jnp.int32, (B, G), 1)).astype(jnp.float32)
    gath = _dot(P, tab_ref[...])
    cs_e = gath[:, :H]
    a_e = gath[:, H:]
    r_e = _dot(P, r_ref[...])
    v = x_ref[...] * r_e + a_e
    ns = _ln(v, gng_ref[...], gnb_ref[...])
    cat = jnp.concatenate([cs_e, ns], axis=1)
    h = _gelu(_dot(_ln(cat, lhg_ref[...], lhb_ref[...]), lhW1_ref[...]) + lhb1_ref[...])
    out_ref[...] = _dot(h, lhW2_ref[...]) + lhb2_ref[...]


def kernel(edge_tokens, question_tokens, edge_batch, selected_mask, selection_order,
           order_emb, type_emb, gn_g, gn_b, qf_g, qf_b, qf_W, qf_bias,
           m1_g, m1_b, m1_W, m1_bias, m2_g, m2_b, m2_W, m2_bias,
           lh_g, lh_b, lh_W1, lh_b1, lh_W2, lh_b2,
           sp_g, sp_b, sp_W1, sp_b1, sp_W2, sp_b2):
    E, H = edge_tokens.shape
    G = question_tokens.shape[0]
    B = _pick_block(E)
    NB = E // B

    row = lambda a: a.reshape(1, -1)
    ordemb = jnp.concatenate(
        [order_emb, jnp.zeros((16 - order_emb.shape[0], H), order_emb.dtype)], axis=0)

    qf2 = pl.pallas_call(
        _qf_kernel,
        out_shape=jax.ShapeDtypeStruct((G, H), jnp.float32),
    )(question_tokens, row(qf_g), row(qf_b), qf_W, row(qf_bias), row(type_emb[2]))

    eb3 = edge_batch.astype(jnp.int32).reshape(NB, 1, B)
    ord3 = selection_order.astype(jnp.int32).reshape(NB, 1, B)
    mask3 = selected_mask.astype(jnp.int32).reshape(NB, 1, B)

    full = lambda shape: pl.BlockSpec(shape, lambda i: (0,) * len(shape))
    idx3 = pl.BlockSpec((1, 1, B), lambda i: (i, 0, 0))
    eblk = pl.BlockSpec((B, H), lambda i: (i, 0))

    x, ssum, cnt = pl.pallas_call(
        _edge_fwd_kernel,
        grid=(NB,),
        in_specs=[eblk, idx3, idx3, idx3, full((G, H)), full((16, H)),
                  full((1, H)), full((1, H)), full((H, H)), full((1, H)),
                  full((1, H)), full((1, H)), full((H, H)), full((1, H))],
        out_specs=[eblk, full((G, H)), full((G, 1))],
        out_shape=[jax.ShapeDtypeStruct((E, H), jnp.float32),
                   jax.ShapeDtypeStruct((G, H), jnp.float32),
                   jax.ShapeDtypeStruct((G, 1), jnp.float32)],
    )(edge_tokens, eb3, ord3, mask3, qf2, ordemb,
      row(m1_g), row(m1_b), m1_W, row(m1_bias),
      row(m2_g), row(m2_b), m2_W, row(m2_bias))

    cs, stop, tab, r = pl.pallas_call(
        _graph_kernel,
        out_shape=[jax.ShapeDtypeStruct((G, H), jnp.float32),
                   jax.ShapeDtypeStruct((G, 1), jnp.float32),
                   jax.ShapeDtypeStruct((G, 2 * H), jnp.float32),
                   jax.ShapeDtypeStruct((G, 1), jnp.float32)],
    )(ssum, cnt, question_tokens, row(gn_g), row(gn_b),
      row(sp_g), row(sp_b), sp_W1, row(sp_b1), sp_W2, sp_b2.reshape(1, 1))

    logits = pl.pallas_call(
        _edge_head_kernel,
        grid=(NB,),
        in_specs=[eblk, idx3, full((G, 2 * H)), full((G, 1)),
                  full((1, H)), full((1, H)),
                  full((1, 2 * H)), full((1, 2 * H)), full((2 * H, H)),
                  full((1, H)), full((H, 1)), full((1, 1))],
        out_specs=pl.BlockSpec((B, 1), lambda i: (i, 0)),
        out_shape=jax.ShapeDtypeStruct((E, 1), jnp.float32),
    )(x, eb3, tab, r, row(gn_g), row(gn_b),
      row(lh_g), row(lh_b), lh_W1, row(lh_b1), lh_W2, lh_b2.reshape(1, 1))

    return logits[:, 0], stop[:, 0], cs
